# baseline (speedup 1.0000x reference)
import functools

import jax
import jax.numpy as jnp
from jax import lax
from jax.experimental import pallas as pl
from jax.experimental.pallas import tpu as pltpu
from jax.experimental.pallas import tpu_sc as plsc

_DIM = 1000
_N0 = 4096
_N1 = 26
_DP = 8
_PIECES_PER_COL = _DIM // _DP
_NPIECES = _N1 * _PIECES_PER_COL
_NW = 32
_MAXK = -(-_NPIECES // _NW)
_GROUPS = _N0 // 16


def _sc_body(idx_hbm, out_hbm, cols, bufs, sems):
    cid = lax.axis_index("c")
    sid = lax.axis_index("s")
    wid = sid * 2 + cid
    iota16 = lax.iota(jnp.int32, 16)
    ones16 = jnp.ones((16,), jnp.float32)
    zeros16 = jnp.zeros((16,), jnp.float32)

    for s in range(2):
        @pl.loop(0, _DP)
        def _zr(r):
            @pl.loop(0, _GROUPS)
            def _zg(g):
                bufs[s, r, pl.ds(g * 16, 16)] = zeros16

    def _scan(slot, piece, val16, mask_only_set):
        d_lo = lax.rem(piece, _PIECES_PER_COL) * _DP

        @pl.loop(0, _GROUPS, unroll=2)
        def _g(g):
            v = cols[slot, pl.ds(g * 16, 16)]
            rel = v - d_lo
            m = (rel >= 0) & (rel < _DP)
            relc = jnp.where(m, rel, 0)
            i_vec = g * 16 + iota16
            plsc.store_scatter(bufs.at[slot], [relc, i_vec], val16, mask=m)

    def _dma(slot, piece):
        c = lax.div(piece, _PIECES_PER_COL)
        d_lo = lax.rem(piece, _PIECES_PER_COL) * _DP
        return pltpu.make_async_copy(
            bufs.at[slot],
            out_hbm.at[c, pl.ds(d_lo, _DP), :],
            sems.at[slot],
        )

    nk = (_NPIECES - wid + _NW - 1) // _NW

    @pl.loop(0, _MAXK)
    def _k(k):
        @pl.when(k < nk)
        def _():
            piece = wid + k * _NW
            slot = lax.rem(k, 2)

            @pl.when(k >= 2)
            def _():
                old = piece - 2 * _NW
                _dma(slot, old).wait()
                _scan(slot, old, zeros16, False)

            c = lax.div(piece, _PIECES_PER_COL)
            pltpu.sync_copy(idx_hbm.at[c], cols.at[slot])
            _scan(slot, piece, ones16, True)
            _dma(slot, piece).start()

    for t in range(2):
        k2 = nk - 2 + t
        piece2 = wid + k2 * _NW
        slot2 = lax.rem(k2, 2)
        _dma(slot2, piece2).wait()


def kernel(tensor):
    n0, n1 = tensor.shape
    idx_t = tensor.astype(jnp.int32).T
    mesh = plsc.VectorSubcoreMesh(core_axis_name="c", subcore_axis_name="s")
    sc_call = functools.partial(
        pl.kernel,
        out_type=jax.ShapeDtypeStruct((_N1, _DIM, _N0), jnp.float32),
        mesh=mesh,
        compiler_params=pltpu.CompilerParams(needs_layout_passes=False),
        scratch_types=[
            pltpu.VMEM((2, _N0), jnp.int32),
            pltpu.VMEM((2, _DP, _N0), jnp.float32),
            pltpu.SemaphoreType.DMA((2,)),
        ],
    )(_sc_body)
    out_phys = sc_call(idx_t)
    return jnp.transpose(out_phys, (2, 0, 1))


# speedup vs baseline: 1.2503x; 1.2503x over previous
import functools

import jax
import jax.numpy as jnp
from jax import lax
from jax.experimental import pallas as pl
from jax.experimental.pallas import tpu as pltpu
from jax.experimental.pallas import tpu_sc as plsc

_DIM = 1000
_N0 = 4096
_N1 = 26
_DP = 8
_PIECES_PER_COL = _DIM // _DP
_NPIECES = _N1 * _PIECES_PER_COL
_NW = 32
_BASE = _NPIECES // _NW
_EXTRA = _NPIECES % _NW
_GROUPS = _N0 // 16


def _sc_body(idx_hbm, out_hbm, cols, bufs, sems):
    cid = lax.axis_index("c")
    sid = lax.axis_index("s")
    wid = sid * 2 + cid
    iota16 = lax.iota(jnp.int32, 16)
    ones16 = jnp.ones((16,), jnp.float32)
    zeros16 = jnp.zeros((16,), jnp.float32)

    lo = wid * _BASE + lax.min(wid, _EXTRA)
    nk = _BASE + jnp.where(wid < _EXTRA, 1, 0)

    for s in range(2):
        @pl.loop(0, _DP)
        def _zr(r):
            @pl.loop(0, _GROUPS)
            def _zg(g):
                bufs[s, r, pl.ds(g * 16, 16)] = zeros16

    def _scan(slot, piece, val16):
        d_lo = lax.rem(piece, _PIECES_PER_COL) * _DP

        @pl.loop(0, _GROUPS, unroll=2)
        def _g(g):
            v = cols[slot, pl.ds(g * 16, 16)]
            rel = v - d_lo
            m = rel.astype(jnp.uint32) < _DP
            relc = jnp.where(m, rel, 0)
            i_vec = g * 16 + iota16
            plsc.store_scatter(bufs.at[slot], [relc, i_vec], val16, mask=m)

    def _dma(slot, piece):
        c = lax.div(piece, _PIECES_PER_COL)
        d_lo = lax.rem(piece, _PIECES_PER_COL) * _DP
        return pltpu.make_async_copy(
            bufs.at[slot],
            out_hbm.at[c, pl.ds(d_lo, _DP), :],
            sems.at[slot],
        )

    @pl.loop(0, nk, init_carry=(jnp.int32(-1), jnp.int32(-1)))
    def _k(k, carry):
        c0, c1 = carry
        piece = lo + k
        slot = lax.rem(k, 2)

        @pl.when(k >= 2)
        def _():
            old = piece - 2
            _dma(slot, old).wait()
            _scan(slot, old, zeros16)

        c = lax.div(piece, _PIECES_PER_COL)
        c_slot = jnp.where(slot == 0, c0, c1)

        @pl.when(c != c_slot)
        def _():
            pltpu.sync_copy(idx_hbm.at[c], cols.at[slot])

        _scan(slot, piece, ones16)
        _dma(slot, piece).start()
        return (
            jnp.where(slot == 0, c, c0),
            jnp.where(slot == 1, c, c1),
        )

    for t in range(2):
        k2 = nk - 2 + t
        piece2 = lo + k2
        slot2 = lax.rem(k2, 2)
        _dma(slot2, piece2).wait()


def kernel(tensor):
    n0, n1 = tensor.shape
    idx_t = tensor.astype(jnp.int32).T
    mesh = plsc.VectorSubcoreMesh(core_axis_name="c", subcore_axis_name="s")
    sc_call = functools.partial(
        pl.kernel,
        out_type=jax.ShapeDtypeStruct((_N1, _DIM, _N0), jnp.float32),
        mesh=mesh,
        compiler_params=pltpu.CompilerParams(needs_layout_passes=False),
        scratch_types=[
            pltpu.VMEM((2, _N0), jnp.int32),
            pltpu.VMEM((2, _DP, _N0), jnp.float32),
            pltpu.SemaphoreType.DMA((2,)),
        ],
    )(_sc_body)
    out_phys = sc_call(idx_t)
    return jnp.transpose(out_phys, (2, 0, 1))


# speedup vs baseline: 4.9917x; 3.9925x over previous
import jax
import jax.numpy as jnp
from jax.experimental import pallas as pl

_DIM = 1000
_R = 1024


def _onehot_body(idx_ref, out_ref):
    idx = idx_ref[...]
    iota = jax.lax.broadcasted_iota(jnp.int32, (1, _DIM, _R), 1)
    out_ref[...] = (iota == idx).astype(jnp.float32)


def kernel(tensor):
    n0, n1 = tensor.shape
    idx_t = tensor.astype(jnp.int32).T.reshape(n1, 1, n0)
    out_phys = pl.pallas_call(
        _onehot_body,
        grid=(n1, n0 // _R),
        in_specs=[pl.BlockSpec((1, 1, _R), lambda c, r: (c, 0, r))],
        out_specs=pl.BlockSpec((1, _DIM, _R), lambda c, r: (c, 0, r)),
        out_shape=jax.ShapeDtypeStruct((n1, _DIM, n0), jnp.float32),
    )(idx_t)
    return jnp.transpose(out_phys, (2, 0, 1))
